# baseline (speedup 1.0000x reference)
import jax
import jax.numpy as jnp
from jax.experimental import pallas as pl

_GROUP_ORDER = 16
_FACTOR = 2
_SUB = _GROUP_ORDER // _FACTOR
_F = 96


def _copy_body(in_ref, out_ref):
    out_ref[...] = in_ref[0]


def kernel(x):
    B, C, H, W = x.shape
    blk_elems = _F * H * W
    rows = blk_elems // 128
    nblocks = B * _SUB
    xv = x.reshape(nblocks, _FACTOR, rows, 128)

    split = 4
    out = pl.pallas_call(
        _copy_body,
        grid=(nblocks, split),
        in_specs=[
            pl.BlockSpec((1, 1, rows // split, 128), lambda i, j: (i, 0, j, 0))
        ],
        out_specs=pl.BlockSpec((1, rows // split, 128), lambda i, j: (i, j, 0)),
        out_shape=jax.ShapeDtypeStruct((nblocks, rows, 128), jnp.float32),
    )(xv)
    return out.reshape(B, _SUB * _F, H, W)
